# baseline (speedup 1.0000x reference)
import functools

import jax
import jax.numpy as jnp
from jax import lax
from jax.experimental import pallas as pl
from jax.experimental.pallas import tpu as pltpu
from jax.experimental.pallas import tpu_sc as plsc

_LANES = 16
_DIM = 64
_CHUNKS = _DIM // _LANES


def _extract0(vec):
    lane = lax.iota(jnp.int32, _LANES)
    return jnp.min(jnp.where(lane == 0, vec, jnp.int32(2147483647)))


def _column(window_ref, col):
    lane = lax.iota(jnp.int32, _LANES)
    cols = jnp.full((_LANES,), col, dtype=jnp.int32)
    return [
        plsc.load_gather(window_ref, [lane + jnp.int32(c * _LANES), cols])
        for c in range(_CHUNKS)
    ]


def _sc_kernel(idx_hbm, nidx_hbm, len_hbm, tt_hbm, out_hbm,
               idx_v, nidx_v, len_v, wa, wb, out_v, sem_a, sem_b):
    wid = lax.axis_index("s") * 2 + lax.axis_index("c")

    @pl.when(wid == 0)
    def _():
        pltpu.sync_copy(idx_hbm.at[pl.ds(0, _LANES)], idx_v)
        pltpu.sync_copy(nidx_hbm.at[pl.ds(0, _LANES)], nidx_v)
        pltpu.sync_copy(len_hbm, len_v)
        ia = _extract0(idx_v[...])
        ib = _extract0(nidx_v[...])
        sa = pl.multiple_of((ia // 128) * 128, 128)
        sb = pl.multiple_of((ib // 128) * 128, 128)
        cp_a = pltpu.async_copy(tt_hbm.at[:, pl.ds(sa, 128)], wa, sem_a)
        cp_b = pltpu.async_copy(tt_hbm.at[:, pl.ds(sb, 128)], wb, sem_b)
        cp_a.wait()
        cp_b.wait()
        a = _column(wa, ia - sa)
        b = _column(wb, ib - sb)
        d = [jnp.abs(x - y) for x, y in zip(a, b)]
        lane = lax.iota(jnp.int32, _LANES)
        d[-1] = jnp.where(lane < 2, d[-1], jnp.float32(jnp.inf))
        m = jnp.minimum(jnp.minimum(d[0], d[1]), jnp.minimum(d[2], d[3]))
        min_d = jnp.min(m)
        ls = len_v[...]
        l1 = jnp.abs(min_d - ls)
        out_v[...] = jnp.where(min_d < ls, jnp.float32(100.0) * l1, l1)
        pltpu.sync_copy(out_v, out_hbm)


def kernel(index_vec, neighbor_index_vec, len_sum, table):
    tt = table.T
    len_arr = jnp.broadcast_to(jnp.reshape(len_sum, (1,)), (_LANES,))
    mesh = plsc.VectorSubcoreMesh(core_axis_name="c", subcore_axis_name="s")
    run = functools.partial(
        pl.kernel,
        out_type=jax.ShapeDtypeStruct((_LANES,), jnp.float32),
        mesh=mesh,
        scratch_types=[
            pltpu.VMEM((_LANES,), jnp.int32),
            pltpu.VMEM((_LANES,), jnp.int32),
            pltpu.VMEM((_LANES,), jnp.float32),
            pltpu.VMEM((_DIM, 128), jnp.float32),
            pltpu.VMEM((_DIM, 128), jnp.float32),
            pltpu.VMEM((_LANES,), jnp.float32),
            pltpu.SemaphoreType.DMA,
            pltpu.SemaphoreType.DMA,
        ],
        compiler_params=pltpu.CompilerParams(use_tc_tiling_on_sc=True,
                                             needs_layout_passes=False),
    )(_sc_kernel)
    out = run(index_vec.astype(jnp.int32), neighbor_index_vec.astype(jnp.int32),
              len_arr, tt)
    return out[0]


# speedup vs baseline: 1.0740x; 1.0740x over previous
import functools

import jax
import jax.numpy as jnp
from jax import lax
from jax.experimental import pallas as pl
from jax.experimental.pallas import tpu as pltpu
from jax.experimental.pallas import tpu_sc as plsc

_LANES = 16
_DIM = 64
_CHUNKS = _DIM // _LANES


def _extract0(vec):
    lane = lax.iota(jnp.int32, _LANES)
    return jnp.min(jnp.where(lane == 0, vec, jnp.int32(2147483647)))


def _column(window_ref, col):
    lane = lax.iota(jnp.int32, _LANES)
    cols = jnp.full((_LANES,), col, dtype=jnp.int32)
    return [
        plsc.load_gather(window_ref, [lane + jnp.int32(c * _LANES), cols])
        for c in range(_CHUNKS)
    ]


def _sc_kernel(idx_hbm, nidx_hbm, len_hbm, tt_hbm, out_hbm,
               idx_v, nidx_v, len_v, wa, wb, out_v, sem_a, sem_b):
    wid = lax.axis_index("s") * 2 + lax.axis_index("c")

    @pl.when(wid == 0)
    def _():
        pltpu.sync_copy(idx_hbm.at[pl.ds(0, _LANES)], idx_v)
        pltpu.sync_copy(nidx_hbm.at[pl.ds(0, _LANES)], nidx_v)
        pltpu.sync_copy(len_hbm, len_v)
        ia = _extract0(idx_v[...])
        ib = _extract0(nidx_v[...])
        sa = pl.multiple_of((ia // 128) * 128, 128)
        sb = pl.multiple_of((ib // 128) * 128, 128)
        cp_a = pltpu.async_copy(tt_hbm.at[:, pl.ds(sa, 128)], wa, sem_a)
        cp_b = pltpu.async_copy(tt_hbm.at[:, pl.ds(sb, 128)], wb, sem_b)
        cp_a.wait()
        cp_b.wait()
        a = _column(wa, ia - sa)
        b = _column(wb, ib - sb)
        d = [jnp.abs(x - y) for x, y in zip(a, b)]
        lane = lax.iota(jnp.int32, _LANES)
        d[-1] = jnp.where(lane < 2, d[-1], jnp.float32(jnp.inf))
        m = jnp.minimum(jnp.minimum(d[0], d[1]), jnp.minimum(d[2], d[3]))
        min_d = jnp.min(m)
        ls = len_v[...]
        l1 = jnp.abs(min_d - ls)
        out_v[...] = jnp.where(min_d < ls, jnp.float32(100.0) * l1, l1)
        pltpu.sync_copy(out_v, out_hbm)


def kernel(index_vec, neighbor_index_vec, len_sum, table):
    tt = table.T
    len_arr = jnp.broadcast_to(jnp.reshape(len_sum, (1,)), (_LANES,))
    mesh = plsc.VectorSubcoreMesh(core_axis_name="c", subcore_axis_name="s", num_cores=1)
    run = functools.partial(
        pl.kernel,
        out_type=jax.ShapeDtypeStruct((_LANES,), jnp.float32),
        mesh=mesh,
        scratch_types=[
            pltpu.VMEM((_LANES,), jnp.int32),
            pltpu.VMEM((_LANES,), jnp.int32),
            pltpu.VMEM((_LANES,), jnp.float32),
            pltpu.VMEM((_DIM, 128), jnp.float32),
            pltpu.VMEM((_DIM, 128), jnp.float32),
            pltpu.VMEM((_LANES,), jnp.float32),
            pltpu.SemaphoreType.DMA,
            pltpu.SemaphoreType.DMA,
        ],
        compiler_params=pltpu.CompilerParams(use_tc_tiling_on_sc=True,
                                             needs_layout_passes=False),
    )(_sc_kernel)
    out = run(index_vec.astype(jnp.int32), neighbor_index_vec.astype(jnp.int32),
              len_arr, tt)
    return out[0]


# speedup vs baseline: 1.1139x; 1.0371x over previous
import functools

import jax
import jax.numpy as jnp
from jax import lax
from jax.experimental import pallas as pl
from jax.experimental.pallas import tpu as pltpu
from jax.experimental.pallas import tpu_sc as plsc

_LANES = 16
_DIM = 64
_CHUNKS = _DIM // _LANES


def _extract0(vec):
    lane = lax.iota(jnp.int32, _LANES)
    return jnp.min(jnp.where(lane == 0, vec, jnp.int32(2147483647)))


def _column(window_ref, col):
    lane = lax.iota(jnp.int32, _LANES)
    cols = jnp.full((_LANES,), col, dtype=jnp.int32)
    return [
        plsc.load_gather(window_ref, [lane + jnp.int32(c * _LANES), cols])
        for c in range(_CHUNKS)
    ]


def _sc_kernel(idx_hbm, nidx_hbm, len_hbm, tt_hbm, out_hbm,
               idx_v, nidx_v, len_v, wa, wb, out_v, sem_i, sem_n, sem_l,
               sem_a, sem_b):
    wid = lax.axis_index("s") + lax.axis_index("c")

    @pl.when(wid == 0)
    def _():
        cp_i = pltpu.async_copy(idx_hbm.at[pl.ds(0, _LANES)], idx_v, sem_i)
        cp_n = pltpu.async_copy(nidx_hbm.at[pl.ds(0, _LANES)], nidx_v, sem_n)
        cp_l = pltpu.async_copy(len_hbm, len_v, sem_l)
        cp_i.wait()
        cp_n.wait()
        ia = _extract0(idx_v[...])
        ib = _extract0(nidx_v[...])
        sa = pl.multiple_of((ia // 128) * 128, 128)
        sb = pl.multiple_of((ib // 128) * 128, 128)
        cp_a = pltpu.async_copy(tt_hbm.at[:, pl.ds(sa, 128)], wa, sem_a)
        cp_b = pltpu.async_copy(tt_hbm.at[:, pl.ds(sb, 128)], wb, sem_b)
        cp_a.wait()
        cp_b.wait()
        a = _column(wa, ia - sa)
        b = _column(wb, ib - sb)
        d = [jnp.abs(x - y) for x, y in zip(a, b)]
        lane = lax.iota(jnp.int32, _LANES)
        d[-1] = jnp.where(lane < 2, d[-1], jnp.float32(jnp.inf))
        m = jnp.minimum(jnp.minimum(d[0], d[1]), jnp.minimum(d[2], d[3]))
        min_d = jnp.min(m)
        cp_l.wait()
        ls = len_v[...]
        l1 = jnp.abs(min_d - ls)
        out_v[...] = jnp.where(min_d < ls, jnp.float32(100.0) * l1, l1)
        pltpu.sync_copy(out_v, out_hbm)


def kernel(index_vec, neighbor_index_vec, len_sum, table):
    tt = table.T
    len_arr = jnp.broadcast_to(jnp.reshape(len_sum, (1,)), (_LANES,))
    mesh = plsc.VectorSubcoreMesh(core_axis_name="c", subcore_axis_name="s",
                                  num_cores=1, num_subcores=1)
    run = functools.partial(
        pl.kernel,
        out_type=jax.ShapeDtypeStruct((_LANES,), jnp.float32),
        mesh=mesh,
        scratch_types=[
            pltpu.VMEM((_LANES,), jnp.int32),
            pltpu.VMEM((_LANES,), jnp.int32),
            pltpu.VMEM((_LANES,), jnp.float32),
            pltpu.VMEM((_DIM, 128), jnp.float32),
            pltpu.VMEM((_DIM, 128), jnp.float32),
            pltpu.VMEM((_LANES,), jnp.float32),
            pltpu.SemaphoreType.DMA,
            pltpu.SemaphoreType.DMA,
            pltpu.SemaphoreType.DMA,
            pltpu.SemaphoreType.DMA,
            pltpu.SemaphoreType.DMA,
        ],
        compiler_params=pltpu.CompilerParams(use_tc_tiling_on_sc=True,
                                             needs_layout_passes=False,
                                             skip_device_barrier=True),
    )(_sc_kernel)
    out = run(index_vec.astype(jnp.int32), neighbor_index_vec.astype(jnp.int32),
              len_arr, tt)
    return out[0]


# speedup vs baseline: 1.1141x; 1.0002x over previous
import functools

import jax
import jax.numpy as jnp
from jax import lax
from jax.experimental import pallas as pl
from jax.experimental.pallas import tpu as pltpu
from jax.experimental.pallas import tpu_sc as plsc

_LANES = 16
_DIM = 64
_CHUNKS = _DIM // _LANES


def _extract0(vec):
    lane = lax.iota(jnp.int32, _LANES)
    return jnp.min(jnp.where(lane == 0, vec, jnp.int32(2147483647)))


def _column(window_ref, col):
    lane = lax.iota(jnp.int32, _LANES)
    cols = jnp.full((_LANES,), col, dtype=jnp.int32)
    return [
        plsc.load_gather(window_ref, [lane + jnp.int32(c * _LANES), cols])
        for c in range(_CHUNKS)
    ]


def _sc_kernel(idx_hbm, nidx_hbm, len_hbm, tt_hbm, out_hbm,
               idx_v, nidx_v, len_v, wa, wb, out_v, sem_i, sem_n, sem_l,
               sem_a, sem_b):
    wid = lax.axis_index("s") + lax.axis_index("c")

    @pl.when(wid == 0)
    def _():
        cp_i = pltpu.async_copy(idx_hbm.at[pl.ds(0, _LANES)], idx_v, sem_i)
        cp_n = pltpu.async_copy(nidx_hbm.at[pl.ds(0, _LANES)], nidx_v, sem_n)
        cp_l = pltpu.async_copy(len_hbm, len_v.at[pl.ds(0, 1)], sem_l)
        cp_i.wait()
        cp_n.wait()
        ia = _extract0(idx_v[...])
        ib = _extract0(nidx_v[...])
        sa = pl.multiple_of((ia // 128) * 128, 128)
        sb = pl.multiple_of((ib // 128) * 128, 128)
        cp_a = pltpu.async_copy(tt_hbm.at[:, pl.ds(sa, 128)], wa, sem_a)
        cp_b = pltpu.async_copy(tt_hbm.at[:, pl.ds(sb, 128)], wb, sem_b)
        cp_a.wait()
        cp_b.wait()
        a = _column(wa, ia - sa)
        b = _column(wb, ib - sb)
        d = [jnp.abs(x - y) for x, y in zip(a, b)]
        lane = lax.iota(jnp.int32, _LANES)
        d[-1] = jnp.where(lane < 2, d[-1], jnp.float32(jnp.inf))
        m = jnp.minimum(jnp.minimum(d[0], d[1]), jnp.minimum(d[2], d[3]))
        min_d = jnp.min(m)
        cp_l.wait()
        lvec = len_v[...]
        ls = jnp.min(jnp.where(lane == 0, lvec, jnp.float32(jnp.inf)))
        l1 = jnp.abs(min_d - ls)
        loss = jnp.where(min_d < ls, jnp.float32(100.0) * l1, l1)
        out_v[...] = jnp.broadcast_to(loss, (_LANES,))
        pltpu.sync_copy(out_v, out_hbm)


def kernel(index_vec, neighbor_index_vec, len_sum, table):
    tt = table.T
    len_arr = jnp.reshape(len_sum, (1,))
    mesh = plsc.VectorSubcoreMesh(core_axis_name="c", subcore_axis_name="s",
                                  num_cores=1, num_subcores=1)
    run = functools.partial(
        pl.kernel,
        out_type=jax.ShapeDtypeStruct((_LANES,), jnp.float32),
        mesh=mesh,
        scratch_types=[
            pltpu.VMEM((_LANES,), jnp.int32),
            pltpu.VMEM((_LANES,), jnp.int32),
            pltpu.VMEM((_LANES,), jnp.float32),
            pltpu.VMEM((_DIM, 128), jnp.float32),
            pltpu.VMEM((_DIM, 128), jnp.float32),
            pltpu.VMEM((_LANES,), jnp.float32),
            pltpu.SemaphoreType.DMA,
            pltpu.SemaphoreType.DMA,
            pltpu.SemaphoreType.DMA,
            pltpu.SemaphoreType.DMA,
            pltpu.SemaphoreType.DMA,
        ],
        compiler_params=pltpu.CompilerParams(use_tc_tiling_on_sc=True,
                                             needs_layout_passes=False,
                                             skip_device_barrier=True,
                                             disable_semaphore_checks=True),
    )(_sc_kernel)
    out = run(index_vec.astype(jnp.int32), neighbor_index_vec.astype(jnp.int32),
              len_arr, tt)
    return out[0]
